# Initial kernel scaffold; baseline (speedup 1.0000x reference)
import jax
import jax.numpy as jnp
from jax.experimental import pallas as pl


def kernel(adj, user_ids, item_ids, user_embed, entity_embed, l0_W1w, l0_W1b, l0_W2w, l0_W2b, l0_Waw, l0_Wab, l0_a, l1_W1w, l1_W1b, l1_W2w, l1_W2b, l1_Waw, l1_Wab, l1_a):
    raise NotImplementedError("write your pallas kernel here")


# speedup vs baseline: 2.6068x; 2.6068x over previous
import jax
import jax.numpy as jnp
from jax import lax
from jax.experimental import pallas as pl
from jax.experimental.pallas import tpu as pltpu
from jax.experimental.pallas import tpu_sc as plsc

N_USERS = 10000
NUM_NODES = 50000
E = 800000
D = 64
B = 1024

NC = 2
NS = 16
L = 16

NPAD = 50048
STRIPE = NPAD // NS
CHUNK = 128
EPAD = 802816
EA = EPAD // (NC * NS)
EB = EPAD // NS
PADNODE = NUM_NODES
PB = B // (NC * NS)

f32 = jnp.float32
i32 = jnp.int32

_MESH = plsc.VectorSubcoreMesh(
    core_axis_name="c", subcore_axis_name="s", num_cores=NC, num_subcores=NS)


def _zero16():
    return jnp.zeros((L,), f32)


def _take16(vec, idx):
    return lax.gather(
        vec, idx[:, None],
        dimension_numbers=lax.GatherDimensionNumbers(
            offset_dims=(), collapsed_slice_dims=(0,), start_index_map=(0,)),
        slice_sizes=(1,),
        mode=lax.GatherScatterMode.PROMISE_IN_BOUNDS)


def _edge_softmax_body(src_hbm, dst_hbm, s_hbm, exp_hbm, dpart_hbm,
                       s_v, sidx, didx, ebuf, dvbuf, denom_sh, _sem):
    c = lax.axis_index("c")
    s = lax.axis_index("s")
    wid = c * NS + s

    pltpu.sync_copy(s_hbm, s_v)

    for j in range(CHUNK // L):
        ebuf[pl.ds(j * L, L)] = _zero16()

    def _zbody(i, carry):
        pltpu.sync_copy(ebuf, denom_sh.at[pl.ds(s * STRIPE + i * CHUNK, CHUNK)])
        return carry
    lax.fori_loop(0, STRIPE // CHUNK, _zbody, 0)
    rem = STRIPE % CHUNK
    if rem:
        pltpu.sync_copy(
            ebuf.at[pl.ds(0, rem)],
            denom_sh.at[pl.ds(s * STRIPE + (STRIPE // CHUNK) * CHUNK, rem)])
    plsc.subcore_barrier()

    base = wid * EA

    def _body(g, carry):
        off = base + g * CHUNK
        pltpu.sync_copy(src_hbm.at[pl.ds(off, CHUNK)], sidx)
        pltpu.sync_copy(dst_hbm.at[pl.ds(off, CHUNK)], didx)
        for j in range(CHUNK // L):
            i1 = sidx[pl.ds(j * L, L)]
            i2 = didx[pl.ds(j * L, L)]
            g1 = plsc.load_gather(s_v, [i1])
            g2 = plsc.load_gather(s_v, [i2 + NPAD])
            x = g1 + g2
            x = jnp.where(x >= 0, x, 0.01 * x)
            ebuf[pl.ds(j * L, L)] = jnp.exp(x)
        pltpu.sync_copy(ebuf, exp_hbm.at[pl.ds(off, CHUNK)])
        pltpu.sync_copy(ebuf, denom_sh.at[didx], add=True)
        return carry
    lax.fori_loop(0, EA // CHUNK, _body, 0)

    plsc.subcore_barrier()
    pltpu.sync_copy(denom_sh.at[pl.ds(s * STRIPE, STRIPE)], dvbuf)
    pltpu.sync_copy(dvbuf, dpart_hbm.at[pl.ds(c * NPAD + s * STRIPE, STRIPE)])


_edge_softmax = pl.kernel(
    _edge_softmax_body,
    out_type=(jax.ShapeDtypeStruct((EPAD,), f32),
              jax.ShapeDtypeStruct((NC * NPAD,), f32)),
    mesh=_MESH,
    compiler_params=pltpu.CompilerParams(needs_layout_passes=False),
    scratch_types=[
        pltpu.VMEM((2 * NPAD,), f32),
        pltpu.VMEM((CHUNK,), i32),
        pltpu.VMEM((CHUNK,), i32),
        pltpu.VMEM((CHUNK,), f32),
        pltpu.VMEM((STRIPE,), f32),
        pltpu.VMEM_SHARED((NPAD,), f32),
        pltpu.SemaphoreType.DMA,
    ],
)


def _aggregate_body(src_hbm, dst_hbm, exp_hbm, msg_hbm, dpart_hbm, z_hbm,
                    hn_hbm, dbuf, sidx, didx0, didx, ebuf, abuf, rows, rows2,
                    acc_sh, gsem):
    c = lax.axis_index("c")
    s = lax.axis_index("s")

    pltpu.sync_copy(dpart_hbm, dbuf)

    def _dinv(i, carry):
        a = dbuf[pl.ds(i * L, L)]
        b = dbuf[pl.ds(NPAD + i * L, L)]
        dbuf[pl.ds(i * L, L)] = 1.0 / (a + b + 1e-9)
        return carry
    lax.fori_loop(0, NPAD // L, _dinv, 0)

    base = s * EB
    rem = STRIPE % CHUNK
    lanes = lax.iota(i32, L)
    hi8 = jnp.where(lanes >= 8, 1, 0)
    lo8 = lanes - 8 * hi8

    for p in range(4):
        q = 4 * c + p
        coff = q * NPAD

        pltpu.sync_copy(z_hbm.at[pl.ds(s * STRIPE, STRIPE)],
                        acc_sh.at[pl.ds(s * STRIPE, STRIPE)])
        plsc.subcore_barrier()

        def _body(g, carry):
            off = base + g * CHUNK
            pltpu.sync_copy(dst_hbm.at[pl.ds(off, CHUNK)], didx0)
            pltpu.sync_copy(src_hbm.at[pl.ds(off, CHUNK)], sidx)
            pltpu.sync_copy(exp_hbm.at[pl.ds(off, CHUNK)], ebuf)
            for j in range(CHUNK // L):
                d16 = didx0[pl.ds(j * L, L)]
                didx[pl.ds(j * L, L)] = d16 + coff
                dinv16 = plsc.load_gather(dbuf, [d16])
                abuf[pl.ds(j * L, L)] = ebuf[pl.ds(j * L, L)] * dinv16
            pltpu.async_copy(msg_hbm.at[didx], rows, gsem).wait()

            def _scale(g2, carry2):
                a16 = abuf[pl.ds(g2 * L, L)]
                for k in range(L // 2):
                    pair = _take16(a16, 2 * k + hi8)
                    ridx = g2 * L + 2 * k + hi8
                    v = plsc.load_gather(rows, [ridx, lo8])
                    plsc.store_scatter(rows, [ridx, lo8], v * pair)
                return carry2
            lax.fori_loop(0, CHUNK // L, _scale, 0)
            pltpu.sync_copy(rows, acc_sh.at[sidx], add=True)
            return carry
        lax.fori_loop(0, EB // CHUNK, _body, 0)

        plsc.subcore_barrier()

        def _drain(i, carry):
            pltpu.sync_copy(acc_sh.at[pl.ds(s * STRIPE + i * CHUNK, CHUNK)],
                            rows2)
            pltpu.sync_copy(
                rows2, hn_hbm.at[pl.ds(coff + s * STRIPE + i * CHUNK, CHUNK)])
            return carry
        lax.fori_loop(0, STRIPE // CHUNK, _drain, 0)
        if rem:
            off3 = (STRIPE // CHUNK) * CHUNK
            pltpu.sync_copy(acc_sh.at[pl.ds(s * STRIPE + off3, rem)],
                            rows2.at[pl.ds(0, rem)])
            pltpu.sync_copy(rows2.at[pl.ds(0, rem)],
                            hn_hbm.at[pl.ds(coff + s * STRIPE + off3, rem)])
        plsc.subcore_barrier()


_aggregate = pl.kernel(
    _aggregate_body,
    out_type=jax.ShapeDtypeStruct((8 * NPAD, 8), f32),
    mesh=_MESH,
    compiler_params=pltpu.CompilerParams(needs_layout_passes=False,
                                         use_tc_tiling_on_sc=False),
    scratch_types=[
        pltpu.VMEM((NC * NPAD,), f32),
        pltpu.VMEM((CHUNK,), i32),
        pltpu.VMEM((CHUNK,), i32),
        pltpu.VMEM((CHUNK,), i32),
        pltpu.VMEM((CHUNK,), f32),
        pltpu.VMEM((CHUNK,), f32),
        pltpu.VMEM((CHUNK, 8), f32),
        pltpu.VMEM((CHUNK, 8), f32),
        pltpu.VMEM_SHARED((NPAD, 8), f32),
        pltpu.SemaphoreType.DMA,
    ],
)


def _score_body(e0_hbm, e1_hbm, e2_hbm, uid_hbm, iid_hbm, out_hbm,
                uix, iix, urows, irows, pacc, obuf, sem):
    c = lax.axis_index("c")
    s = lax.axis_index("s")
    wid = c * NS + s
    base = wid * PB
    pltpu.sync_copy(uid_hbm.at[pl.ds(base, PB)], uix)
    pltpu.sync_copy(iid_hbm.at[pl.ds(base, PB)], iix)
    for j in range(PB // L):
        iix[pl.ds(j * L, L)] = iix[pl.ds(j * L, L)] + N_USERS
    for i in range(PB):
        pacc[i, pl.ds(0, L)] = _zero16()
    for tab in (e0_hbm, e1_hbm, e2_hbm):
        pltpu.async_copy(tab.at[uix], urows, sem).wait()
        pltpu.async_copy(tab.at[iix], irows, sem).wait()
        for i in range(PB):
            acc = urows[i, pl.ds(0, L)] * irows[i, pl.ds(0, L)]
            for q in range(1, D // L):
                acc = acc + urows[i, pl.ds(q * L, L)] * irows[i, pl.ds(q * L, L)]
            pacc[i, pl.ds(0, L)] = pacc[i, pl.ds(0, L)] + acc
    lanes = lax.iota(i32, L)
    for g2 in range(PB // L):
        ovec = _zero16()
        for k in range(L):
            tot = jnp.sum(pacc[g2 * L + k, pl.ds(0, L)])
            ovec = jnp.where(lanes == k, tot, ovec)
        obuf[pl.ds(g2 * L, L)] = ovec
    pltpu.sync_copy(obuf, out_hbm.at[pl.ds(base, PB)])


_score = pl.kernel(
    _score_body,
    out_type=jax.ShapeDtypeStruct((B,), f32),
    mesh=_MESH,
    compiler_params=pltpu.CompilerParams(needs_layout_passes=False,
                                         use_tc_tiling_on_sc=False),
    scratch_types=[
        pltpu.VMEM((PB,), i32),
        pltpu.VMEM((PB,), i32),
        pltpu.VMEM((PB, D), f32),
        pltpu.VMEM((PB, D), f32),
        pltpu.VMEM((PB, L), f32),
        pltpu.VMEM((PB,), f32),
        pltpu.SemaphoreType.DMA,
    ],
)


_RB = 2176


def _dense1_body(x_ref, w1w_ref, w1b_ref, waw_ref, wab_ref, a_ref,
                 s_ref, m_ref):
    x = x_ref[...]
    a = a_ref[...]
    a2col = jnp.concatenate([a[:D, :], a[D:, :]], axis=1)
    v = lax.dot_general(waw_ref[...], a2col, (((0,), (0,)), ((), ())))
    cstT = lax.dot_general(a2col, wab_ref[...], (((0,), (1,)), ((), ())))
    sT = lax.dot_general(v, x, (((0,), (1,)), ((), ())),
                         preferred_element_type=f32)
    s_ref[...] = sT + cstT
    m_ref[...] = lax.dot_general(x, w1w_ref[...], (((1,), (1,)), ((), ())),
                                 preferred_element_type=f32) + w1b_ref[0]


def _dense1(X, W1w, W1b, Waw, Wab, a):
    nb = NPAD // _RB
    full = lambda shape: pl.BlockSpec(shape, lambda i, q: (0, 0))
    s_tab, msg = pl.pallas_call(
        _dense1_body,
        grid=(nb, 8),
        in_specs=[
            pl.BlockSpec((_RB, D), lambda i, q: (i, 0)),
            pl.BlockSpec((8, D), lambda i, q: (q, 0)),
            pl.BlockSpec((1, 1, 8), lambda i, q: (q, 0, 0)),
            full((D, D)),
            full((1, D)),
            full((2 * D, 1)),
        ],
        out_specs=[
            pl.BlockSpec((2, _RB), lambda i, q: (0, i)),
            pl.BlockSpec((_RB, 8), lambda i, q: (q * nb + i, 0)),
        ],
        out_shape=[
            jax.ShapeDtypeStruct((2, NPAD), f32),
            jax.ShapeDtypeStruct((8 * NPAD, 8), f32),
        ],
    )(X, W1w, W1b.reshape(8, 1, 8), Waw, Wab.reshape(1, D), a)
    return s_tab.reshape(-1), msg


def _dense2_body(x_ref, h0, h1, h2, h3, h4, h5, h6, h7, w2w_ref, w2b_ref,
                 y_ref):
    x = x_ref[...]
    hn = jnp.concatenate(
        [h0[...], h1[...], h2[...], h3[...],
         h4[...], h5[...], h6[...], h7[...]], axis=1)
    t = x + hn + lax.dot_general(x * hn, w2w_ref[...], (((1,), (1,)), ((), ())),
                                 preferred_element_type=f32) + w2b_ref[...]
    y = jnp.where(t >= 0, t, 0.01 * t)
    n = jnp.sqrt(jnp.sum(y * y, axis=1, keepdims=True))
    y_ref[...] = y / jnp.maximum(n, 1e-12)


def _dense2(X, hn, W2w, W2b):
    nb = NPAD // _RB
    hspec = lambda q: pl.BlockSpec((_RB, 8), lambda i, q=q: (i + q * nb, 0))
    return pl.pallas_call(
        _dense2_body,
        grid=(nb,),
        in_specs=[
            pl.BlockSpec((_RB, D), lambda i: (i, 0)),
            hspec(0), hspec(1), hspec(2), hspec(3),
            hspec(4), hspec(5), hspec(6), hspec(7),
            pl.BlockSpec((D, D), lambda i: (0, 0)),
            pl.BlockSpec((1, D), lambda i: (0, 0)),
        ],
        out_specs=pl.BlockSpec((_RB, D), lambda i: (i, 0)),
        out_shape=jax.ShapeDtypeStruct((NPAD, D), f32),
    )(X, hn, hn, hn, hn, hn, hn, hn, hn, W2w, W2b.reshape(1, D))


def _layer(X, src, dst, zrows, W1w, W1b, W2w, W2b, Waw, Wab, a):
    s_flat, msg = _dense1(X, W1w, W1b, Waw, Wab, a)
    expv, dpart = _edge_softmax(src, dst, s_flat)
    hn = _aggregate(src, dst, expv, msg, dpart, zrows)
    return _dense2(X, hn, W2w, W2b)


def kernel(adj, user_ids, item_ids, user_embed, entity_embed,
           l0_W1w, l0_W1b, l0_W2w, l0_W2b, l0_Waw, l0_Wab, l0_a,
           l1_W1w, l1_W1b, l1_W2w, l1_W2b, l1_Waw, l1_Wab, l1_a):
    X0 = jnp.concatenate(
        [user_embed, entity_embed,
         jnp.zeros((NPAD - NUM_NODES, D), f32)], axis=0)
    epad = jnp.full((EPAD - E,), PADNODE, i32)
    src = jnp.concatenate([adj[0], epad])
    dst = jnp.concatenate([adj[1], epad])
    zrows = jnp.zeros((NPAD, 8), f32)

    ws = (jnp.stack([l0_W1w, l1_W1w]), jnp.stack([l0_W1b, l1_W1b]),
          jnp.stack([l0_W2w, l1_W2w]), jnp.stack([l0_W2b, l1_W2b]),
          jnp.stack([l0_Waw, l1_Waw]), jnp.stack([l0_Wab, l1_Wab]),
          jnp.stack([l0_a, l1_a]))
    nlayers = lax.optimization_barrier(jnp.int32(2))

    def _cond(st):
        return st[0] < nlayers

    def _step(st):
        i, X, e1 = st
        w = jax.tree.map(lambda t: lax.dynamic_index_in_dim(t, i, 0, False), ws)
        Y = _layer(X, src, dst, zrows, *w)
        e1 = lax.select(i == 0, Y, e1)
        return (i + 1, Y, e1)

    _, X2, X1 = lax.while_loop(_cond, _step, (jnp.int32(0), X0, X0))
    return _score(X0, X1, X2, user_ids, item_ids)


# speedup vs baseline: 4.0107x; 1.5386x over previous
import jax
import jax.numpy as jnp
from jax import lax
from jax.experimental import pallas as pl
from jax.experimental.pallas import tpu as pltpu
from jax.experimental.pallas import tpu_sc as plsc

N_USERS = 10000
NUM_NODES = 50000
E = 800000
D = 64
B = 1024

NC = 2
NS = 16
L = 16

NPAD = 50048
STRIPE = NPAD // NS
CHUNK = 128
EPAD = 802816
EA = EPAD // (NC * NS)
EB = EPAD // NS
PADNODE = NUM_NODES
PB = B // (NC * NS)
BIGF = 8
BIGN = BIGF * CHUNK

f32 = jnp.float32
i32 = jnp.int32

_MESH = plsc.VectorSubcoreMesh(
    core_axis_name="c", subcore_axis_name="s", num_cores=NC, num_subcores=NS)


def _zero16():
    return jnp.zeros((L,), f32)


def _take16(vec, idx):
    return lax.gather(
        vec, idx[:, None],
        dimension_numbers=lax.GatherDimensionNumbers(
            offset_dims=(), collapsed_slice_dims=(0,), start_index_map=(0,)),
        slice_sizes=(1,),
        mode=lax.GatherScatterMode.PROMISE_IN_BOUNDS)


def _edge_softmax_body(src_hbm, dst_hbm, s_hbm, exp_hbm, dpart_hbm,
                       s_v, sidx, didx, ebuf, dvbuf, denom_sh, _sem):
    c = lax.axis_index("c")
    s = lax.axis_index("s")
    wid = c * NS + s

    pltpu.sync_copy(s_hbm, s_v)

    for j in range(CHUNK // L):
        ebuf[pl.ds(j * L, L)] = _zero16()

    def _zbody(i, carry):
        pltpu.sync_copy(ebuf, denom_sh.at[pl.ds(s * STRIPE + i * CHUNK, CHUNK)])
        return carry
    lax.fori_loop(0, STRIPE // CHUNK, _zbody, 0)
    rem = STRIPE % CHUNK
    if rem:
        pltpu.sync_copy(
            ebuf.at[pl.ds(0, rem)],
            denom_sh.at[pl.ds(s * STRIPE + (STRIPE // CHUNK) * CHUNK, rem)])
    plsc.subcore_barrier()

    base = wid * EA

    def _body(g, carry):
        off = base + g * CHUNK
        pltpu.sync_copy(src_hbm.at[pl.ds(off, CHUNK)], sidx)
        pltpu.sync_copy(dst_hbm.at[pl.ds(off, CHUNK)], didx)
        for j in range(CHUNK // L):
            i1 = sidx[pl.ds(j * L, L)]
            i2 = didx[pl.ds(j * L, L)]
            g1 = plsc.load_gather(s_v, [i1])
            g2 = plsc.load_gather(s_v, [i2 + NPAD])
            x = g1 + g2
            x = jnp.where(x >= 0, x, 0.01 * x)
            ebuf[pl.ds(j * L, L)] = jnp.exp(x)
        pltpu.sync_copy(ebuf, exp_hbm.at[pl.ds(off, CHUNK)])
        pltpu.sync_copy(ebuf, denom_sh.at[didx], add=True)
        return carry
    lax.fori_loop(0, EA // CHUNK, _body, 0)

    plsc.subcore_barrier()
    pltpu.sync_copy(denom_sh.at[pl.ds(s * STRIPE, STRIPE)], dvbuf)
    pltpu.sync_copy(dvbuf, dpart_hbm.at[pl.ds(c * NPAD + s * STRIPE, STRIPE)])


_edge_softmax = pl.kernel(
    _edge_softmax_body,
    out_type=(jax.ShapeDtypeStruct((EPAD,), f32),
              jax.ShapeDtypeStruct((NC * NPAD,), f32)),
    mesh=_MESH,
    compiler_params=pltpu.CompilerParams(needs_layout_passes=False),
    scratch_types=[
        pltpu.VMEM((2 * NPAD,), f32),
        pltpu.VMEM((CHUNK,), i32),
        pltpu.VMEM((CHUNK,), i32),
        pltpu.VMEM((CHUNK,), f32),
        pltpu.VMEM((STRIPE,), f32),
        pltpu.VMEM_SHARED((NPAD,), f32),
        pltpu.SemaphoreType.DMA,
    ],
)


def _aggregate_body(src_hbm, dst_hbm, exp_hbm, msg_hbm, dpart_hbm, z_hbm,
                    hn_hbm, dbuf, sidx, dstb, srcb, didx, ebuf, abuf, rows,
                    rows2, acc_sh, gsem):
    c = lax.axis_index("c")
    s = lax.axis_index("s")

    pltpu.sync_copy(dpart_hbm, dbuf)

    def _dinv(i, carry):
        a = dbuf[pl.ds(i * L, L)]
        b = dbuf[pl.ds(NPAD + i * L, L)]
        dbuf[pl.ds(i * L, L)] = 1.0 / (a + b + 1e-9)
        return carry
    lax.fori_loop(0, NPAD // L, _dinv, 0)

    base = s * EB
    rem = STRIPE % CHUNK
    lanes = lax.iota(i32, L)
    hi8 = jnp.where(lanes >= 8, 1, 0)
    lo8 = lanes - 8 * hi8

    for p in range(4):
        q = 4 * c + p
        coff = q * NPAD

        pltpu.sync_copy(z_hbm.at[pl.ds(s * STRIPE, STRIPE)],
                        acc_sh.at[pl.ds(s * STRIPE, STRIPE)])
        plsc.subcore_barrier()

        def _body(g, carry):
            r = lax.rem(g, jnp.int32(BIGF))

            @pl.when(r == 0)
            def _load_big():
                off = base + g * CHUNK
                pltpu.sync_copy(dst_hbm.at[pl.ds(off, BIGN)], dstb)
                pltpu.sync_copy(src_hbm.at[pl.ds(off, BIGN)], srcb)
                pltpu.sync_copy(exp_hbm.at[pl.ds(off, BIGN)], ebuf)

            rb = r * CHUNK
            for j in range(CHUNK // L):
                d16 = dstb[pl.ds(rb + j * L, L)]
                didx[pl.ds(j * L, L)] = d16 + coff
                sidx[pl.ds(j * L, L)] = srcb[pl.ds(rb + j * L, L)]
                dinv16 = plsc.load_gather(dbuf, [d16])
                abuf[pl.ds(j * L, L)] = ebuf[pl.ds(rb + j * L, L)] * dinv16
            pltpu.async_copy(msg_hbm.at[didx], rows, gsem).wait()

            def _scale(g2, carry2):
                a16 = abuf[pl.ds(g2 * L, L)]
                for k in range(L // 2):
                    pair = _take16(a16, 2 * k + hi8)
                    ridx = g2 * L + 2 * k + hi8
                    v = plsc.load_gather(rows, [ridx, lo8])
                    plsc.store_scatter(rows, [ridx, lo8], v * pair)
                return carry2
            lax.fori_loop(0, CHUNK // L, _scale, 0)
            pltpu.sync_copy(rows, acc_sh.at[sidx], add=True)
            return carry
        lax.fori_loop(0, EB // CHUNK, _body, 0)

        plsc.subcore_barrier()

        def _drain(i, carry):
            pltpu.sync_copy(acc_sh.at[pl.ds(s * STRIPE + i * CHUNK, CHUNK)],
                            rows2)
            pltpu.sync_copy(
                rows2, hn_hbm.at[pl.ds(coff + s * STRIPE + i * CHUNK, CHUNK)])
            return carry
        lax.fori_loop(0, STRIPE // CHUNK, _drain, 0)
        if rem:
            off3 = (STRIPE // CHUNK) * CHUNK
            pltpu.sync_copy(acc_sh.at[pl.ds(s * STRIPE + off3, rem)],
                            rows2.at[pl.ds(0, rem)])
            pltpu.sync_copy(rows2.at[pl.ds(0, rem)],
                            hn_hbm.at[pl.ds(coff + s * STRIPE + off3, rem)])
        plsc.subcore_barrier()


_aggregate = pl.kernel(
    _aggregate_body,
    out_type=jax.ShapeDtypeStruct((8 * NPAD, 8), f32),
    mesh=_MESH,
    compiler_params=pltpu.CompilerParams(needs_layout_passes=False,
                                         use_tc_tiling_on_sc=False),
    scratch_types=[
        pltpu.VMEM((NC * NPAD,), f32),
        pltpu.VMEM((CHUNK,), i32),
        pltpu.VMEM((BIGN,), i32),
        pltpu.VMEM((BIGN,), i32),
        pltpu.VMEM((CHUNK,), i32),
        pltpu.VMEM((BIGN,), f32),
        pltpu.VMEM((CHUNK,), f32),
        pltpu.VMEM((CHUNK, 8), f32),
        pltpu.VMEM((CHUNK, 8), f32),
        pltpu.VMEM_SHARED((NPAD, 8), f32),
        pltpu.SemaphoreType.DMA,
    ],
)


def _score_body(e0_hbm, e1_hbm, e2_hbm, uid_hbm, iid_hbm, out_hbm,
                uix, iix, urows, irows, pacc, obuf, sem):
    c = lax.axis_index("c")
    s = lax.axis_index("s")
    wid = c * NS + s
    base = wid * PB
    pltpu.sync_copy(uid_hbm.at[pl.ds(base, PB)], uix)
    pltpu.sync_copy(iid_hbm.at[pl.ds(base, PB)], iix)
    for j in range(PB // L):
        iix[pl.ds(j * L, L)] = iix[pl.ds(j * L, L)] + N_USERS
    for i in range(PB):
        pacc[i, pl.ds(0, L)] = _zero16()
    for tab in (e0_hbm, e1_hbm, e2_hbm):
        pltpu.async_copy(tab.at[uix], urows, sem).wait()
        pltpu.async_copy(tab.at[iix], irows, sem).wait()
        for i in range(PB):
            acc = urows[i, pl.ds(0, L)] * irows[i, pl.ds(0, L)]
            for q in range(1, D // L):
                acc = acc + urows[i, pl.ds(q * L, L)] * irows[i, pl.ds(q * L, L)]
            pacc[i, pl.ds(0, L)] = pacc[i, pl.ds(0, L)] + acc
    lanes = lax.iota(i32, L)
    for g2 in range(PB // L):
        ovec = _zero16()
        for k in range(L):
            tot = jnp.sum(pacc[g2 * L + k, pl.ds(0, L)])
            ovec = jnp.where(lanes == k, tot, ovec)
        obuf[pl.ds(g2 * L, L)] = ovec
    pltpu.sync_copy(obuf, out_hbm.at[pl.ds(base, PB)])


_score = pl.kernel(
    _score_body,
    out_type=jax.ShapeDtypeStruct((B,), f32),
    mesh=_MESH,
    compiler_params=pltpu.CompilerParams(needs_layout_passes=False,
                                         use_tc_tiling_on_sc=False),
    scratch_types=[
        pltpu.VMEM((PB,), i32),
        pltpu.VMEM((PB,), i32),
        pltpu.VMEM((PB, D), f32),
        pltpu.VMEM((PB, D), f32),
        pltpu.VMEM((PB, L), f32),
        pltpu.VMEM((PB,), f32),
        pltpu.SemaphoreType.DMA,
    ],
)


_RB = 2176


def _dense1_body(x_ref, w1w_ref, w1b_ref, waw_ref, wab_ref, a_ref,
                 s_ref, m_ref):
    x = x_ref[...]
    a = a_ref[...]
    a2col = jnp.concatenate([a[:D, :], a[D:, :]], axis=1)
    v = lax.dot_general(waw_ref[...], a2col, (((0,), (0,)), ((), ())))
    cstT = lax.dot_general(a2col, wab_ref[...], (((0,), (1,)), ((), ())))
    sT = lax.dot_general(v, x, (((0,), (1,)), ((), ())),
                         preferred_element_type=f32)
    s_ref[...] = sT + cstT
    m_ref[...] = lax.dot_general(x, w1w_ref[...], (((1,), (1,)), ((), ())),
                                 preferred_element_type=f32) + w1b_ref[0]


def _dense1(X, W1w, W1b, Waw, Wab, a):
    nb = NPAD // _RB
    full = lambda shape: pl.BlockSpec(shape, lambda i, q: (0, 0))
    s_tab, msg = pl.pallas_call(
        _dense1_body,
        grid=(nb, 8),
        in_specs=[
            pl.BlockSpec((_RB, D), lambda i, q: (i, 0)),
            pl.BlockSpec((8, D), lambda i, q: (q, 0)),
            pl.BlockSpec((1, 1, 8), lambda i, q: (q, 0, 0)),
            full((D, D)),
            full((1, D)),
            full((2 * D, 1)),
        ],
        out_specs=[
            pl.BlockSpec((2, _RB), lambda i, q: (0, i)),
            pl.BlockSpec((_RB, 8), lambda i, q: (q * nb + i, 0)),
        ],
        out_shape=[
            jax.ShapeDtypeStruct((2, NPAD), f32),
            jax.ShapeDtypeStruct((8 * NPAD, 8), f32),
        ],
    )(X, W1w, W1b.reshape(8, 1, 8), Waw, Wab.reshape(1, D), a)
    return s_tab.reshape(-1), msg


def _dense2_body(x_ref, h0, h1, h2, h3, h4, h5, h6, h7, w2w_ref, w2b_ref,
                 y_ref):
    x = x_ref[...]
    hn = jnp.concatenate(
        [h0[...], h1[...], h2[...], h3[...],
         h4[...], h5[...], h6[...], h7[...]], axis=1)
    t = x + hn + lax.dot_general(x * hn, w2w_ref[...], (((1,), (1,)), ((), ())),
                                 preferred_element_type=f32) + w2b_ref[...]
    y = jnp.where(t >= 0, t, 0.01 * t)
    n = jnp.sqrt(jnp.sum(y * y, axis=1, keepdims=True))
    y_ref[...] = y / jnp.maximum(n, 1e-12)


def _dense2(X, hn, W2w, W2b):
    nb = NPAD // _RB
    hspec = lambda q: pl.BlockSpec((_RB, 8), lambda i, q=q: (i + q * nb, 0))
    return pl.pallas_call(
        _dense2_body,
        grid=(nb,),
        in_specs=[
            pl.BlockSpec((_RB, D), lambda i: (i, 0)),
            hspec(0), hspec(1), hspec(2), hspec(3),
            hspec(4), hspec(5), hspec(6), hspec(7),
            pl.BlockSpec((D, D), lambda i: (0, 0)),
            pl.BlockSpec((1, D), lambda i: (0, 0)),
        ],
        out_specs=pl.BlockSpec((_RB, D), lambda i: (i, 0)),
        out_shape=jax.ShapeDtypeStruct((NPAD, D), f32),
    )(X, hn, hn, hn, hn, hn, hn, hn, hn, W2w, W2b.reshape(1, D))


def _layer(X, src, dst, zrows, W1w, W1b, W2w, W2b, Waw, Wab, a):
    s_flat, msg = _dense1(X, W1w, W1b, Waw, Wab, a)
    expv, dpart = _edge_softmax(src, dst, s_flat)
    hn = _aggregate(src, dst, expv, msg, dpart, zrows)
    return _dense2(X, hn, W2w, W2b)


def kernel(adj, user_ids, item_ids, user_embed, entity_embed,
           l0_W1w, l0_W1b, l0_W2w, l0_W2b, l0_Waw, l0_Wab, l0_a,
           l1_W1w, l1_W1b, l1_W2w, l1_W2b, l1_Waw, l1_Wab, l1_a):
    X0 = jnp.concatenate(
        [user_embed, entity_embed,
         jnp.zeros((NPAD - NUM_NODES, D), f32)], axis=0)
    epad = jnp.full((EPAD - E,), PADNODE, i32)
    src = jnp.concatenate([adj[0], epad])
    dst = jnp.concatenate([adj[1], epad])
    zrows = jnp.zeros((NPAD, 8), f32)

    ws = (jnp.stack([l0_W1w, l1_W1w]), jnp.stack([l0_W1b, l1_W1b]),
          jnp.stack([l0_W2w, l1_W2w]), jnp.stack([l0_W2b, l1_W2b]),
          jnp.stack([l0_Waw, l1_Waw]), jnp.stack([l0_Wab, l1_Wab]),
          jnp.stack([l0_a, l1_a]))
    nlayers = lax.optimization_barrier(jnp.int32(2))

    def _cond(st):
        return st[0] < nlayers

    def _step(st):
        i, X, e1 = st
        w = jax.tree.map(lambda t: lax.dynamic_index_in_dim(t, i, 0, False), ws)
        Y = _layer(X, src, dst, zrows, *w)
        e1 = lax.select(i == 0, Y, e1)
        return (i + 1, Y, e1)

    _, X2, X1 = lax.while_loop(_cond, _step, (jnp.int32(0), X0, X0))
    return _score(X0, X1, X2, user_ids, item_ids)


# speedup vs baseline: 4.1598x; 1.0372x over previous
import jax
import jax.numpy as jnp
from jax import lax
from jax.experimental import pallas as pl
from jax.experimental.pallas import tpu as pltpu
from jax.experimental.pallas import tpu_sc as plsc

N_USERS = 10000
NUM_NODES = 50000
E = 800000
D = 64
B = 1024

NC = 2
NS = 16
L = 16

NPAD = 50048
STRIPE = NPAD // NS
CHUNK = 128
EPAD = 802816
EA = EPAD // (NC * NS)
EB = EPAD // NS
PADNODE = NUM_NODES
PB = B // (NC * NS)
BIGF = 8
BIGN = BIGF * CHUNK
DRN = 136
DVN = 3136

f32 = jnp.float32
i32 = jnp.int32

_MESH = plsc.VectorSubcoreMesh(
    core_axis_name="c", subcore_axis_name="s", num_cores=NC, num_subcores=NS)


def _zero16():
    return jnp.zeros((L,), f32)


def _take16(vec, idx):
    return lax.gather(
        vec, idx[:, None],
        dimension_numbers=lax.GatherDimensionNumbers(
            offset_dims=(), collapsed_slice_dims=(0,), start_index_map=(0,)),
        slice_sizes=(1,),
        mode=lax.GatherScatterMode.PROMISE_IN_BOUNDS)


def _edge_softmax_body(src_hbm, dst_hbm, s_hbm, exp_hbm, dpart_hbm,
                       s_v, sidx, didx, ebuf, dvbuf, denom_sh, _sem):
    c = lax.axis_index("c")
    s = lax.axis_index("s")
    wid = c * NS + s

    pltpu.sync_copy(s_hbm, s_v)

    for j in range(CHUNK // L):
        ebuf[pl.ds(j * L, L)] = _zero16()

    def _zbody(i, carry):
        pltpu.sync_copy(ebuf, denom_sh.at[pl.ds(s * STRIPE + i * CHUNK, CHUNK)])
        return carry
    lax.fori_loop(0, STRIPE // CHUNK, _zbody, 0)
    rem = STRIPE % CHUNK
    if rem:
        pltpu.sync_copy(
            ebuf.at[pl.ds(0, rem)],
            denom_sh.at[pl.ds(s * STRIPE + (STRIPE // CHUNK) * CHUNK, rem)])
    plsc.subcore_barrier()

    base = wid * EA

    def _body(g, carry):
        off = base + g * CHUNK
        pltpu.sync_copy(src_hbm.at[pl.ds(off, CHUNK)], sidx)
        pltpu.sync_copy(dst_hbm.at[pl.ds(off, CHUNK)], didx)
        for j in range(CHUNK // L):
            i1 = sidx[pl.ds(j * L, L)]
            i2 = didx[pl.ds(j * L, L)]
            g1 = plsc.load_gather(s_v, [i1])
            g2 = plsc.load_gather(s_v, [i2 + NPAD])
            x = g1 + g2
            x = jnp.where(x >= 0, x, 0.01 * x)
            ebuf[pl.ds(j * L, L)] = jnp.exp(x)
        pltpu.sync_copy(ebuf, exp_hbm.at[pl.ds(off, CHUNK)])
        pltpu.sync_copy(ebuf, denom_sh.at[didx], add=True)
        return carry
    lax.fori_loop(0, EA // CHUNK, _body, 0)

    plsc.subcore_barrier()
    pltpu.sync_copy(denom_sh.at[pl.ds(s * STRIPE, STRIPE)], dvbuf)
    pltpu.sync_copy(dvbuf, dpart_hbm.at[pl.ds(c * NPAD + s * STRIPE, STRIPE)])


_edge_softmax = pl.kernel(
    _edge_softmax_body,
    out_type=(jax.ShapeDtypeStruct((EPAD,), f32),
              jax.ShapeDtypeStruct((NC * NPAD,), f32)),
    mesh=_MESH,
    compiler_params=pltpu.CompilerParams(needs_layout_passes=False),
    scratch_types=[
        pltpu.VMEM((2 * NPAD,), f32),
        pltpu.VMEM((CHUNK,), i32),
        pltpu.VMEM((CHUNK,), i32),
        pltpu.VMEM((CHUNK,), f32),
        pltpu.VMEM((STRIPE,), f32),
        pltpu.VMEM_SHARED((NPAD,), f32),
        pltpu.SemaphoreType.DMA,
    ],
)


def _aggregate_body(src_hbm, dst_hbm, exp_hbm, msg_hbm, dpart_hbm, z_hbm,
                    hn_hbm, dbuf, sidx, dstb, srcb, didx, ebuf, abuf, rows,
                    rows2, acc_sh, gsem):
    c = lax.axis_index("c")
    s = lax.axis_index("s")

    pltpu.sync_copy(dpart_hbm, dbuf)

    def _dinv(i, carry):
        a = dbuf[pl.ds(i * L, L)]
        b = dbuf[pl.ds(NPAD + i * L, L)]
        dbuf[pl.ds(i * L, L)] = 1.0 / (a + b + 1e-9)
        return carry
    lax.fori_loop(0, NPAD // L, _dinv, 0)

    base = s * EB
    rem = STRIPE % CHUNK
    lanes = lax.iota(i32, L)
    hi8 = jnp.where(lanes >= 8, 1, 0)
    lo8 = lanes - 8 * hi8

    for p in range(4):
        q = 4 * c + p
        coff = q * NPAD

        pltpu.sync_copy(z_hbm.at[pl.ds(s * STRIPE, STRIPE)],
                        acc_sh.at[pl.ds(s * STRIPE, STRIPE)])
        plsc.subcore_barrier()

        def _body(g, carry):
            r = lax.rem(g, jnp.int32(BIGF))

            @pl.when(r == 0)
            def _load_big():
                off = base + g * CHUNK
                pltpu.sync_copy(dst_hbm.at[pl.ds(off, BIGN)], dstb)
                pltpu.sync_copy(src_hbm.at[pl.ds(off, BIGN)], srcb)
                pltpu.sync_copy(exp_hbm.at[pl.ds(off, BIGN)], ebuf)

            rb = r * CHUNK
            for j in range(CHUNK // L):
                d16 = dstb[pl.ds(rb + j * L, L)]
                didx[pl.ds(j * L, L)] = d16 + coff
            desc = pltpu.async_copy(msg_hbm.at[didx], rows, gsem)
            for j in range(CHUNK // L):
                d16 = dstb[pl.ds(rb + j * L, L)]
                sidx[pl.ds(j * L, L)] = srcb[pl.ds(rb + j * L, L)]
                dinv16 = plsc.load_gather(dbuf, [d16])
                abuf[pl.ds(j * L, L)] = ebuf[pl.ds(rb + j * L, L)] * dinv16
            desc.wait()

            def _scale(g2, carry2):
                a16 = abuf[pl.ds(g2 * L, L)]
                for kk in range(L // 2):
                    pair = _take16(a16, 2 * kk + hi8)
                    ridx = g2 * L + 2 * kk + hi8
                    v = plsc.load_gather(rows, [ridx, lo8])
                    plsc.store_scatter(rows, [ridx, lo8], v * pair)
                return carry2
            lax.fori_loop(0, CHUNK // L, _scale, 0)
            pltpu.sync_copy(rows, acc_sh.at[sidx], add=True)
            return carry
        lax.fori_loop(0, EB // CHUNK, _body, 0)

        plsc.subcore_barrier()

        def _drain(i, carry):
            pltpu.sync_copy(acc_sh.at[pl.ds(s * STRIPE + i * CHUNK, CHUNK)],
                            rows2)
            pltpu.sync_copy(
                rows2, hn_hbm.at[pl.ds(coff + s * STRIPE + i * CHUNK, CHUNK)])
            return carry
        lax.fori_loop(0, STRIPE // CHUNK, _drain, 0)
        if rem:
            off3 = (STRIPE // CHUNK) * CHUNK
            pltpu.sync_copy(acc_sh.at[pl.ds(s * STRIPE + off3, rem)],
                            rows2.at[pl.ds(0, rem)])
            pltpu.sync_copy(rows2.at[pl.ds(0, rem)],
                            hn_hbm.at[pl.ds(coff + s * STRIPE + off3, rem)])
        plsc.subcore_barrier()


_aggregate = pl.kernel(
    _aggregate_body,
    out_type=jax.ShapeDtypeStruct((8 * NPAD, 8), f32),
    mesh=_MESH,
    compiler_params=pltpu.CompilerParams(needs_layout_passes=False,
                                         use_tc_tiling_on_sc=False),
    scratch_types=[
        pltpu.VMEM((NC * NPAD,), f32),
        pltpu.VMEM((CHUNK,), i32),
        pltpu.VMEM((BIGN,), i32),
        pltpu.VMEM((BIGN,), i32),
        pltpu.VMEM((CHUNK,), i32),
        pltpu.VMEM((BIGN,), f32),
        pltpu.VMEM((CHUNK,), f32),
        pltpu.VMEM((CHUNK, 8), f32),
        pltpu.VMEM((CHUNK, 8), f32),
        pltpu.VMEM_SHARED((NPAD, 8), f32),
        pltpu.SemaphoreType.DMA,
    ],
)


def _score_body(e0_hbm, e1_hbm, e2_hbm, uid_hbm, iid_hbm, out_hbm,
                uix, iix, urows, irows, pacc, obuf, sem):
    c = lax.axis_index("c")
    s = lax.axis_index("s")
    wid = c * NS + s
    base = wid * PB
    pltpu.sync_copy(uid_hbm.at[pl.ds(base, PB)], uix)
    pltpu.sync_copy(iid_hbm.at[pl.ds(base, PB)], iix)
    for j in range(PB // L):
        iix[pl.ds(j * L, L)] = iix[pl.ds(j * L, L)] + N_USERS
    for i in range(PB):
        pacc[i, pl.ds(0, L)] = _zero16()
    for tab in (e0_hbm, e1_hbm, e2_hbm):
        pltpu.async_copy(tab.at[uix], urows, sem).wait()
        pltpu.async_copy(tab.at[iix], irows, sem).wait()
        for i in range(PB):
            acc = urows[i, pl.ds(0, L)] * irows[i, pl.ds(0, L)]
            for q in range(1, D // L):
                acc = acc + urows[i, pl.ds(q * L, L)] * irows[i, pl.ds(q * L, L)]
            pacc[i, pl.ds(0, L)] = pacc[i, pl.ds(0, L)] + acc
    lanes = lax.iota(i32, L)
    for g2 in range(PB // L):
        ovec = _zero16()
        for k in range(L):
            tot = jnp.sum(pacc[g2 * L + k, pl.ds(0, L)])
            ovec = jnp.where(lanes == k, tot, ovec)
        obuf[pl.ds(g2 * L, L)] = ovec
    pltpu.sync_copy(obuf, out_hbm.at[pl.ds(base, PB)])


_score = pl.kernel(
    _score_body,
    out_type=jax.ShapeDtypeStruct((B,), f32),
    mesh=_MESH,
    compiler_params=pltpu.CompilerParams(needs_layout_passes=False,
                                         use_tc_tiling_on_sc=False),
    scratch_types=[
        pltpu.VMEM((PB,), i32),
        pltpu.VMEM((PB,), i32),
        pltpu.VMEM((PB, D), f32),
        pltpu.VMEM((PB, D), f32),
        pltpu.VMEM((PB, L), f32),
        pltpu.VMEM((PB,), f32),
        pltpu.SemaphoreType.DMA,
    ],
)


_RB = 2176


def _dense1_body(x_ref, w1w_ref, w1b_ref, waw_ref, wab_ref, a_ref,
                 s_ref, m_ref):
    x = x_ref[...]
    a = a_ref[...]
    a2col = jnp.concatenate([a[:D, :], a[D:, :]], axis=1)
    v = lax.dot_general(waw_ref[...], a2col, (((0,), (0,)), ((), ())))
    cstT = lax.dot_general(a2col, wab_ref[...], (((0,), (1,)), ((), ())))
    sT = lax.dot_general(v, x, (((0,), (1,)), ((), ())),
                         preferred_element_type=f32)
    s_ref[...] = sT + cstT
    m_ref[...] = lax.dot_general(x, w1w_ref[...], (((1,), (1,)), ((), ())),
                                 preferred_element_type=f32) + w1b_ref[0]


def _dense1(X, W1w, W1b, Waw, Wab, a):
    nb = NPAD // _RB
    full = lambda shape: pl.BlockSpec(shape, lambda i, q: (0, 0))
    s_tab, msg = pl.pallas_call(
        _dense1_body,
        grid=(nb, 8),
        in_specs=[
            pl.BlockSpec((_RB, D), lambda i, q: (i, 0)),
            pl.BlockSpec((8, D), lambda i, q: (q, 0)),
            pl.BlockSpec((1, 1, 8), lambda i, q: (q, 0, 0)),
            full((D, D)),
            full((1, D)),
            full((2 * D, 1)),
        ],
        out_specs=[
            pl.BlockSpec((2, _RB), lambda i, q: (0, i)),
            pl.BlockSpec((_RB, 8), lambda i, q: (q * nb + i, 0)),
        ],
        out_shape=[
            jax.ShapeDtypeStruct((2, NPAD), f32),
            jax.ShapeDtypeStruct((8 * NPAD, 8), f32),
        ],
    )(X, W1w, W1b.reshape(8, 1, 8), Waw, Wab.reshape(1, D), a)
    return s_tab.reshape(-1), msg


def _dense2_body(x_ref, h0, h1, h2, h3, h4, h5, h6, h7, w2w_ref, w2b_ref,
                 y_ref):
    x = x_ref[...]
    hn = jnp.concatenate(
        [h0[...], h1[...], h2[...], h3[...],
         h4[...], h5[...], h6[...], h7[...]], axis=1)
    t = x + hn + lax.dot_general(x * hn, w2w_ref[...], (((1,), (1,)), ((), ())),
                                 preferred_element_type=f32) + w2b_ref[...]
    y = jnp.where(t >= 0, t, 0.01 * t)
    n = jnp.sqrt(jnp.sum(y * y, axis=1, keepdims=True))
    y_ref[...] = y / jnp.maximum(n, 1e-12)


def _dense2(X, hn, W2w, W2b):
    nb = NPAD // _RB
    hspec = lambda q: pl.BlockSpec((_RB, 8), lambda i, q=q: (i + q * nb, 0))
    return pl.pallas_call(
        _dense2_body,
        grid=(nb,),
        in_specs=[
            pl.BlockSpec((_RB, D), lambda i: (i, 0)),
            hspec(0), hspec(1), hspec(2), hspec(3),
            hspec(4), hspec(5), hspec(6), hspec(7),
            pl.BlockSpec((D, D), lambda i: (0, 0)),
            pl.BlockSpec((1, D), lambda i: (0, 0)),
        ],
        out_specs=pl.BlockSpec((_RB, D), lambda i: (i, 0)),
        out_shape=jax.ShapeDtypeStruct((NPAD, D), f32),
    )(X, hn, hn, hn, hn, hn, hn, hn, hn, W2w, W2b.reshape(1, D))


def _layer(X, src, dst, zrows, W1w, W1b, W2w, W2b, Waw, Wab, a):
    s_flat, msg = _dense1(X, W1w, W1b, Waw, Wab, a)
    expv, dpart = _edge_softmax(src, dst, s_flat)
    hn = _aggregate(src, dst, expv, msg, dpart, zrows)
    return _dense2(X, hn, W2w, W2b)


def kernel(adj, user_ids, item_ids, user_embed, entity_embed,
           l0_W1w, l0_W1b, l0_W2w, l0_W2b, l0_Waw, l0_Wab, l0_a,
           l1_W1w, l1_W1b, l1_W2w, l1_W2b, l1_Waw, l1_Wab, l1_a):
    X0 = jnp.concatenate(
        [user_embed, entity_embed,
         jnp.zeros((NPAD - NUM_NODES, D), f32)], axis=0)
    epad = jnp.full((EPAD - E,), PADNODE, i32)
    src = jnp.concatenate([adj[0], epad])
    dst = jnp.concatenate([adj[1], epad])
    zrows = jnp.zeros((NPAD, 8), f32)

    ws = (jnp.stack([l0_W1w, l1_W1w]), jnp.stack([l0_W1b, l1_W1b]),
          jnp.stack([l0_W2w, l1_W2w]), jnp.stack([l0_W2b, l1_W2b]),
          jnp.stack([l0_Waw, l1_Waw]), jnp.stack([l0_Wab, l1_Wab]),
          jnp.stack([l0_a, l1_a]))
    nlayers = lax.optimization_barrier(jnp.int32(2))

    def _cond(st):
        return st[0] < nlayers

    def _step(st):
        i, X, e1 = st
        w = jax.tree.map(lambda t: lax.dynamic_index_in_dim(t, i, 0, False), ws)
        Y = _layer(X, src, dst, zrows, *w)
        e1 = lax.select(i == 0, Y, e1)
        return (i + 1, Y, e1)

    _, X2, X1 = lax.while_loop(_cond, _step, (jnp.int32(0), X0, X0))
    return _score(X0, X1, X2, user_ids, item_ids)


# speedup vs baseline: 4.3368x; 1.0425x over previous
import jax
import jax.numpy as jnp
from jax import lax
from jax.experimental import pallas as pl
from jax.experimental.pallas import tpu as pltpu
from jax.experimental.pallas import tpu_sc as plsc

N_USERS = 10000
NUM_NODES = 50000
E = 800000
D = 64
B = 1024

NC = 2
NS = 16
L = 16

NPAD = 50048
STRIPE = NPAD // NS
CHUNK = 128
EPAD = 802816
EA = EPAD // (NC * NS)
EB = EPAD // NS
PADNODE = NUM_NODES
PB = B // (NC * NS)
BIGF = 8
BIGN = BIGF * CHUNK
DRN = 136
DVN = 3136

f32 = jnp.float32
i32 = jnp.int32

_MESH = plsc.VectorSubcoreMesh(
    core_axis_name="c", subcore_axis_name="s", num_cores=NC, num_subcores=NS)


def _zero16():
    return jnp.zeros((L,), f32)


def _take16(vec, idx):
    return lax.gather(
        vec, idx[:, None],
        dimension_numbers=lax.GatherDimensionNumbers(
            offset_dims=(), collapsed_slice_dims=(0,), start_index_map=(0,)),
        slice_sizes=(1,),
        mode=lax.GatherScatterMode.PROMISE_IN_BOUNDS)


def _edge_softmax_body(src_hbm, dst_hbm, s_hbm, exp_hbm, dpart_hbm,
                       s_v, srcb, dstb, didx, ebuf, dvbuf, denom_sh, _sem):
    c = lax.axis_index("c")
    s = lax.axis_index("s")
    wid = c * NS + s

    pltpu.sync_copy(s_hbm, s_v)

    for j in range(CHUNK // L):
        ebuf[pl.ds(j * L, L)] = _zero16()

    def _zbody(i, carry):
        pltpu.sync_copy(ebuf, denom_sh.at[pl.ds(s * STRIPE + i * CHUNK, CHUNK)])
        return carry
    lax.fori_loop(0, STRIPE // CHUNK, _zbody, 0)
    rem = STRIPE % CHUNK
    if rem:
        pltpu.sync_copy(
            ebuf.at[pl.ds(0, rem)],
            denom_sh.at[pl.ds(s * STRIPE + (STRIPE // CHUNK) * CHUNK, rem)])
    plsc.subcore_barrier()

    base = wid * EA

    def _body(g, carry):
        off = base + g * CHUNK
        r = lax.rem(g, jnp.int32(BIGF))

        @pl.when(r == 0)
        def _load_big():
            pltpu.sync_copy(src_hbm.at[pl.ds(off, BIGN)], srcb)
            pltpu.sync_copy(dst_hbm.at[pl.ds(off, BIGN)], dstb)

        rb = r * CHUNK
        for j in range(CHUNK // L):
            i1 = srcb[pl.ds(rb + j * L, L)]
            i2 = dstb[pl.ds(rb + j * L, L)]
            didx[pl.ds(j * L, L)] = i2
            g1 = plsc.load_gather(s_v, [i1])
            g2 = plsc.load_gather(s_v, [i2 + NPAD])
            x = g1 + g2
            x = jnp.where(x >= 0, x, 0.01 * x)
            ebuf[pl.ds(j * L, L)] = jnp.exp(x)
        pltpu.sync_copy(ebuf, exp_hbm.at[pl.ds(off, CHUNK)])
        pltpu.sync_copy(ebuf, denom_sh.at[didx], add=True)
        return carry
    lax.fori_loop(0, EA // CHUNK, _body, 0)

    plsc.subcore_barrier()
    pltpu.sync_copy(denom_sh.at[pl.ds(s * STRIPE, STRIPE)], dvbuf)
    pltpu.sync_copy(dvbuf, dpart_hbm.at[pl.ds(c * NPAD + s * STRIPE, STRIPE)])


_edge_softmax = pl.kernel(
    _edge_softmax_body,
    out_type=(jax.ShapeDtypeStruct((EPAD,), f32),
              jax.ShapeDtypeStruct((NC * NPAD,), f32)),
    mesh=_MESH,
    compiler_params=pltpu.CompilerParams(needs_layout_passes=False),
    scratch_types=[
        pltpu.VMEM((2 * NPAD,), f32),
        pltpu.VMEM((BIGN,), i32),
        pltpu.VMEM((BIGN,), i32),
        pltpu.VMEM((CHUNK,), i32),
        pltpu.VMEM((CHUNK,), f32),
        pltpu.VMEM((STRIPE,), f32),
        pltpu.VMEM_SHARED((NPAD,), f32),
        pltpu.SemaphoreType.DMA,
    ],
)


def _aggregate_body(src_hbm, dst_hbm, exp_hbm, msg_hbm, dpart_hbm, z_hbm,
                    hn_hbm, dbuf, sidx, dstb, srcb, didx, ebuf, abuf, rows,
                    rows2, acc_sh, gsem):
    c = lax.axis_index("c")
    s = lax.axis_index("s")

    pltpu.sync_copy(dpart_hbm, dbuf)

    def _dinv(i, carry):
        a = dbuf[pl.ds(i * L, L)]
        b = dbuf[pl.ds(NPAD + i * L, L)]
        dbuf[pl.ds(i * L, L)] = 1.0 / (a + b + 1e-9)
        return carry
    lax.fori_loop(0, NPAD // L, _dinv, 0)

    base = s * EB
    rem = STRIPE % CHUNK
    lanes = lax.iota(i32, L)
    hi8 = jnp.where(lanes >= 8, 1, 0)
    lo8 = lanes - 8 * hi8

    for p in range(4):
        q = 4 * c + p
        coff = q * NPAD

        pltpu.sync_copy(z_hbm.at[pl.ds(s * STRIPE, STRIPE)],
                        acc_sh.at[pl.ds(s * STRIPE, STRIPE)])
        plsc.subcore_barrier()

        def _body(g, carry):
            r = lax.rem(g, jnp.int32(BIGF))

            @pl.when(r == 0)
            def _load_big():
                off = base + g * CHUNK
                pltpu.sync_copy(dst_hbm.at[pl.ds(off, BIGN)], dstb)
                pltpu.sync_copy(src_hbm.at[pl.ds(off, BIGN)], srcb)
                pltpu.sync_copy(exp_hbm.at[pl.ds(off, BIGN)], ebuf)

            rb = r * CHUNK
            for j in range(CHUNK // L):
                d16 = dstb[pl.ds(rb + j * L, L)]
                didx[pl.ds(j * L, L)] = d16 + coff
            desc = pltpu.async_copy(msg_hbm.at[didx], rows, gsem)
            for j in range(CHUNK // L):
                d16 = dstb[pl.ds(rb + j * L, L)]
                sidx[pl.ds(j * L, L)] = srcb[pl.ds(rb + j * L, L)]
                dinv16 = plsc.load_gather(dbuf, [d16])
                abuf[pl.ds(j * L, L)] = ebuf[pl.ds(rb + j * L, L)] * dinv16
            desc.wait()

            def _scale(g2, carry2):
                a16 = abuf[pl.ds(g2 * L, L)]
                for kk in range(L // 2):
                    pair = _take16(a16, 2 * kk + hi8)
                    ridx = g2 * L + 2 * kk + hi8
                    v = plsc.load_gather(rows, [ridx, lo8])
                    plsc.store_scatter(rows, [ridx, lo8], v * pair)
                return carry2
            lax.fori_loop(0, CHUNK // L, _scale, 0)
            pltpu.sync_copy(rows, acc_sh.at[sidx], add=True)
            return carry
        lax.fori_loop(0, EB // CHUNK, _body, 0)

        plsc.subcore_barrier()

        def _drain(i, carry):
            pltpu.sync_copy(acc_sh.at[pl.ds(s * STRIPE + i * CHUNK, CHUNK)],
                            rows2)
            pltpu.sync_copy(
                rows2, hn_hbm.at[pl.ds(coff + s * STRIPE + i * CHUNK, CHUNK)])
            return carry
        lax.fori_loop(0, STRIPE // CHUNK, _drain, 0)
        if rem:
            off3 = (STRIPE // CHUNK) * CHUNK
            pltpu.sync_copy(acc_sh.at[pl.ds(s * STRIPE + off3, rem)],
                            rows2.at[pl.ds(0, rem)])
            pltpu.sync_copy(rows2.at[pl.ds(0, rem)],
                            hn_hbm.at[pl.ds(coff + s * STRIPE + off3, rem)])
        plsc.subcore_barrier()


_aggregate = pl.kernel(
    _aggregate_body,
    out_type=jax.ShapeDtypeStruct((8 * NPAD, 8), f32),
    mesh=_MESH,
    compiler_params=pltpu.CompilerParams(needs_layout_passes=False,
                                         use_tc_tiling_on_sc=False),
    scratch_types=[
        pltpu.VMEM((NC * NPAD,), f32),
        pltpu.VMEM((CHUNK,), i32),
        pltpu.VMEM((BIGN,), i32),
        pltpu.VMEM((BIGN,), i32),
        pltpu.VMEM((CHUNK,), i32),
        pltpu.VMEM((BIGN,), f32),
        pltpu.VMEM((CHUNK,), f32),
        pltpu.VMEM((CHUNK, 8), f32),
        pltpu.VMEM((CHUNK, 8), f32),
        pltpu.VMEM_SHARED((NPAD, 8), f32),
        pltpu.SemaphoreType.DMA,
    ],
)


def _score_body(e0_hbm, e1_hbm, e2_hbm, uid_hbm, iid_hbm, out_hbm,
                uix, iix, urows, irows, pacc, obuf, sem):
    c = lax.axis_index("c")
    s = lax.axis_index("s")
    wid = c * NS + s
    base = wid * PB
    pltpu.sync_copy(uid_hbm.at[pl.ds(base, PB)], uix)
    pltpu.sync_copy(iid_hbm.at[pl.ds(base, PB)], iix)
    for j in range(PB // L):
        iix[pl.ds(j * L, L)] = iix[pl.ds(j * L, L)] + N_USERS
    for i in range(PB):
        pacc[i, pl.ds(0, L)] = _zero16()
    for tab in (e0_hbm, e1_hbm, e2_hbm):
        pltpu.async_copy(tab.at[uix], urows, sem).wait()
        pltpu.async_copy(tab.at[iix], irows, sem).wait()
        for i in range(PB):
            acc = urows[i, pl.ds(0, L)] * irows[i, pl.ds(0, L)]
            for q in range(1, D // L):
                acc = acc + urows[i, pl.ds(q * L, L)] * irows[i, pl.ds(q * L, L)]
            pacc[i, pl.ds(0, L)] = pacc[i, pl.ds(0, L)] + acc
    lanes = lax.iota(i32, L)
    for g2 in range(PB // L):
        ovec = _zero16()
        for k in range(L):
            tot = jnp.sum(pacc[g2 * L + k, pl.ds(0, L)])
            ovec = jnp.where(lanes == k, tot, ovec)
        obuf[pl.ds(g2 * L, L)] = ovec
    pltpu.sync_copy(obuf, out_hbm.at[pl.ds(base, PB)])


_score = pl.kernel(
    _score_body,
    out_type=jax.ShapeDtypeStruct((B,), f32),
    mesh=_MESH,
    compiler_params=pltpu.CompilerParams(needs_layout_passes=False,
                                         use_tc_tiling_on_sc=False),
    scratch_types=[
        pltpu.VMEM((PB,), i32),
        pltpu.VMEM((PB,), i32),
        pltpu.VMEM((PB, D), f32),
        pltpu.VMEM((PB, D), f32),
        pltpu.VMEM((PB, L), f32),
        pltpu.VMEM((PB,), f32),
        pltpu.SemaphoreType.DMA,
    ],
)


_RB = 2176


def _dense1_body(x_ref, w1w_ref, w1b_ref, waw_ref, wab_ref, a_ref,
                 s_ref, m_ref):
    x = x_ref[...]
    a = a_ref[...]
    a2col = jnp.concatenate([a[:D, :], a[D:, :]], axis=1)
    v = lax.dot_general(waw_ref[...], a2col, (((0,), (0,)), ((), ())))
    cstT = lax.dot_general(a2col, wab_ref[...], (((0,), (1,)), ((), ())))
    sT = lax.dot_general(v, x, (((0,), (1,)), ((), ())),
                         preferred_element_type=f32)
    s_ref[...] = sT + cstT
    m_ref[...] = lax.dot_general(x, w1w_ref[...], (((1,), (1,)), ((), ())),
                                 preferred_element_type=f32) + w1b_ref[0]


def _dense1(X, W1w, W1b, Waw, Wab, a):
    nb = NPAD // _RB
    full = lambda shape: pl.BlockSpec(shape, lambda i, q: (0, 0))
    s_tab, msg = pl.pallas_call(
        _dense1_body,
        grid=(nb, 8),
        in_specs=[
            pl.BlockSpec((_RB, D), lambda i, q: (i, 0)),
            pl.BlockSpec((8, D), lambda i, q: (q, 0)),
            pl.BlockSpec((1, 1, 8), lambda i, q: (q, 0, 0)),
            full((D, D)),
            full((1, D)),
            full((2 * D, 1)),
        ],
        out_specs=[
            pl.BlockSpec((2, _RB), lambda i, q: (0, i)),
            pl.BlockSpec((_RB, 8), lambda i, q: (q * nb + i, 0)),
        ],
        out_shape=[
            jax.ShapeDtypeStruct((2, NPAD), f32),
            jax.ShapeDtypeStruct((8 * NPAD, 8), f32),
        ],
    )(X, W1w, W1b.reshape(8, 1, 8), Waw, Wab.reshape(1, D), a)
    return s_tab.reshape(-1), msg


def _dense2_body(x_ref, h0, h1, h2, h3, h4, h5, h6, h7, w2w_ref, w2b_ref,
                 y_ref):
    x = x_ref[...]
    hn = jnp.concatenate(
        [h0[...], h1[...], h2[...], h3[...],
         h4[...], h5[...], h6[...], h7[...]], axis=1)
    t = x + hn + lax.dot_general(x * hn, w2w_ref[...], (((1,), (1,)), ((), ())),
                                 preferred_element_type=f32) + w2b_ref[...]
    y = jnp.where(t >= 0, t, 0.01 * t)
    n = jnp.sqrt(jnp.sum(y * y, axis=1, keepdims=True))
    y_ref[...] = y / jnp.maximum(n, 1e-12)


def _dense2(X, hn, W2w, W2b):
    nb = NPAD // _RB
    hspec = lambda q: pl.BlockSpec((_RB, 8), lambda i, q=q: (i + q * nb, 0))
    return pl.pallas_call(
        _dense2_body,
        grid=(nb,),
        in_specs=[
            pl.BlockSpec((_RB, D), lambda i: (i, 0)),
            hspec(0), hspec(1), hspec(2), hspec(3),
            hspec(4), hspec(5), hspec(6), hspec(7),
            pl.BlockSpec((D, D), lambda i: (0, 0)),
            pl.BlockSpec((1, D), lambda i: (0, 0)),
        ],
        out_specs=pl.BlockSpec((_RB, D), lambda i: (i, 0)),
        out_shape=jax.ShapeDtypeStruct((NPAD, D), f32),
    )(X, hn, hn, hn, hn, hn, hn, hn, hn, W2w, W2b.reshape(1, D))


def _layer(X, src, dst, zrows, W1w, W1b, W2w, W2b, Waw, Wab, a):
    s_flat, msg = _dense1(X, W1w, W1b, Waw, Wab, a)
    expv, dpart = _edge_softmax(src, dst, s_flat)
    hn = _aggregate(src, dst, expv, msg, dpart, zrows)
    return _dense2(X, hn, W2w, W2b)


def kernel(adj, user_ids, item_ids, user_embed, entity_embed,
           l0_W1w, l0_W1b, l0_W2w, l0_W2b, l0_Waw, l0_Wab, l0_a,
           l1_W1w, l1_W1b, l1_W2w, l1_W2b, l1_Waw, l1_Wab, l1_a):
    X0 = jnp.concatenate(
        [user_embed, entity_embed,
         jnp.zeros((NPAD - NUM_NODES, D), f32)], axis=0)
    epad = jnp.full((EPAD - E,), PADNODE, i32)
    src = jnp.concatenate([adj[0], epad])
    dst = jnp.concatenate([adj[1], epad])
    zrows = jnp.zeros((NPAD, 8), f32)

    ws = (jnp.stack([l0_W1w, l1_W1w]), jnp.stack([l0_W1b, l1_W1b]),
          jnp.stack([l0_W2w, l1_W2w]), jnp.stack([l0_W2b, l1_W2b]),
          jnp.stack([l0_Waw, l1_Waw]), jnp.stack([l0_Wab, l1_Wab]),
          jnp.stack([l0_a, l1_a]))
    nlayers = lax.optimization_barrier(jnp.int32(2))

    def _cond(st):
        return st[0] < nlayers

    def _step(st):
        i, X, e1 = st
        w = jax.tree.map(lambda t: lax.dynamic_index_in_dim(t, i, 0, False), ws)
        Y = _layer(X, src, dst, zrows, *w)
        e1 = lax.select(i == 0, Y, e1)
        return (i + 1, Y, e1)

    _, X2, X1 = lax.while_loop(_cond, _step, (jnp.int32(0), X0, X0))
    return _score(X0, X1, X2, user_ids, item_ids)
